# baseline (speedup 1.0000x reference)
import functools

import jax
import jax.numpy as jnp
from jax import lax
from jax.experimental import pallas as pl
from jax.experimental.pallas import tpu as pltpu
from jax.experimental.pallas import tpu_sc as plsc

N = 320000
D = 128
NUM_MOL = 1024


NSTREAM = 2
ROWS = 6400


def _mlp_body(*refs):
    x_refs = refs[:NSTREAM]
    w1t_ref, b1c_ref, w2r_ref, b2_ref, o_ref = refs[NSTREAM:]
    for s, x_ref in enumerate(x_refs):
        xt = x_ref[...].T
        ht = jnp.dot(w1t_ref[...], xt, preferred_element_type=jnp.float32)
        ht = ht + b1c_ref[...]
        ht = ht * jax.nn.sigmoid(ht)
        et = jnp.dot(w2r_ref[...], ht, preferred_element_type=jnp.float32)
        o_ref[0, s, 0, :] = (et + b2_ref[0, 0])[0]


def _mlp(x, w1t, b1c, w2r, b2r, grid, block_off):
    return pl.pallas_call(
        _mlp_body,
        grid=(grid,),
        in_specs=[
            pl.BlockSpec(
                (ROWS, D),
                functools.partial(
                    lambda s, i: (block_off + i * NSTREAM + s, 0), s))
            for s in range(NSTREAM)
        ] + [
            pl.BlockSpec((D, D), lambda i: (0, 0)),
            pl.BlockSpec((D, 1), lambda i: (0, 0)),
            pl.BlockSpec((1, D), lambda i: (0, 0)),
            pl.BlockSpec((1, 1), lambda i: (0, 0)),
        ],
        out_specs=pl.BlockSpec((1, NSTREAM, 1, ROWS), lambda i: (i, 0, 0, 0)),
        out_shape=jax.ShapeDtypeStruct((grid, NSTREAM, 1, ROWS), jnp.float32),
        compiler_params=pltpu.CompilerParams(
            dimension_semantics=("arbitrary",),
        ),
    )(*([x] * NSTREAM), w1t, b1c, w2r, b2r)


NC = 2
NS = 16
NW = NC * NS
STRIP = 128

@functools.cache
def _make_segsum(n, gb_off):
    chunk = n // NW
    mesh = plsc.VectorSubcoreMesh(
        core_axis_name="c", subcore_axis_name="s",
        num_cores=NC, num_subcores=NS,
    )
    return functools.partial(
        pl.kernel,
        mesh=mesh,
        out_type=jax.ShapeDtypeStruct((NC, NUM_MOL), jnp.float32),
        scratch_types=[
            pltpu.VMEM((chunk,), jnp.float32),
            pltpu.VMEM((chunk + 16,), jnp.int32),
            pltpu.VMEM((NUM_MOL,), jnp.float32),
            pltpu.VMEM_SHARED((NS, NUM_MOL), jnp.float32),
            pltpu.VMEM((NS, STRIP), jnp.float32),
            pltpu.VMEM((STRIP,), jnp.float32),
            pltpu.SemaphoreType.DMA,
            pltpu.SemaphoreType.DMA,
        ],
        compiler_params=pltpu.CompilerParams(
            needs_layout_passes=False,
            disable_bounds_checks=True,
            disable_semaphore_checks=True,
        ),
    )(functools.partial(_segsum_body, chunk, gb_off))


def _segsum_body(CHUNK, gb_off, e_hbm, gb_hbm, out_hbm, e_buf, ids_buf, acc,
                 shared, red_buf, strip_buf, sem_e, sem_g):
    NVEC = CHUNK // 16
    cid = lax.axis_index("c")
    sid = lax.axis_index("s")
    wid = cid * NS + sid
    start = wid * CHUNK

    iota = lax.iota(jnp.int32, 16)
    sent_idx = jnp.where(iota < 8, iota, CHUNK + iota)
    plsc.store_scatter(ids_buf, [sent_idx], jnp.full((16,), -1, jnp.int32))

    cp_e = pltpu.async_copy(e_hbm.at[pl.ds(start, CHUNK)], e_buf, sem_e)
    cp_g = pltpu.async_copy(gb_hbm.at[pl.ds(gb_off + start, CHUNK)],
                            ids_buf.at[pl.ds(8, CHUNK)], sem_g)

    zeros16 = jnp.zeros((16,), jnp.float32)

    def zero_body(k, _):
        acc[pl.ds(k * 16, 16)] = zeros16
        return 0

    lax.fori_loop(0, NUM_MOL // 16, zero_body, 0)
    cp_e.wait()
    cp_g.wait()

    UNROLL = 5

    def _main(kk, carry):
        base0 = kk * (16 * UNROLL)
        for u in range(UNROLL):
            base = base0 + u * 16
            e_v = e_buf[pl.ds(base, 16)]
            ids = ids_buf[pl.ds(base + 8, 16)]
            prev = ids_buf[pl.ds(base + 7, 16)]
            nxt = ids_buf[pl.ds(base + 9, 16)]
            c = plsc.cumsum(e_v) + carry
            plsc.addupdate_scatter(acc, [ids], c, mask=ids != nxt)
            plsc.addupdate_scatter(acc, [ids], e_v - c, mask=ids != prev)
            carry = carry + jnp.sum(e_v)
        return carry

    lax.fori_loop(0, NVEC // UNROLL, _main, jnp.float32(0.0))

    pltpu.sync_copy(acc, shared.at[sid])
    plsc.subcore_barrier()

    @pl.when(sid < NUM_MOL // STRIP)
    def _():
        pltpu.sync_copy(shared.at[:, pl.ds(sid * STRIP, STRIP)], red_buf)

        def red_body(j, _):
            s = red_buf[0, pl.ds(j * 16, 16)]
            for t in range(1, NS):
                s = s + red_buf[t, pl.ds(j * 16, 16)]
            strip_buf[pl.ds(j * 16, 16)] = s
            return 0

        lax.fori_loop(0, STRIP // 16, red_body, 0)
        pltpu.sync_copy(strip_buf, out_hbm.at[cid, pl.ds(sid * STRIP, STRIP)])


N_A = 192000
N_B = N - N_A


def kernel(atoms_h, graph_batch, W1, b1, W2, b2):
    w1t = W1.T
    b1c = b1.reshape(D, 1).astype(jnp.float32)
    w2r = W2.reshape(1, D).astype(jnp.float32)
    b2r = b2.reshape(1, 1).astype(jnp.float32)
    gb = graph_batch.astype(jnp.int32)

    grid_a = N_A // (NSTREAM * ROWS)
    grid_b = N_B // (NSTREAM * ROWS)
    e_a = _mlp(atoms_h, w1t, b1c, w2r, b2r, grid_a, 0).reshape(N_A)
    e_b = _mlp(atoms_h, w1t, b1c, w2r, b2r, grid_b,
               NSTREAM * grid_a).reshape(N_B)
    p_a = _make_segsum(N_A, 0)(e_a, gb)
    p_b = _make_segsum(N_B, N_A)(e_b, gb)
    return p_a[0] + p_a[1] + p_b[0] + p_b[1]


# speedup vs baseline: 1.0092x; 1.0092x over previous
import functools

import jax
import jax.numpy as jnp
from jax import lax
from jax.experimental import pallas as pl
from jax.experimental.pallas import tpu as pltpu
from jax.experimental.pallas import tpu_sc as plsc

N = 320000
D = 128
NUM_MOL = 1024


NSTREAM = 2
ROWS = 6400


def _mlp_body(*refs):
    x_refs = refs[:NSTREAM]
    w1t_ref, b1c_ref, w2r_ref, b2_ref, o_ref = refs[NSTREAM:]
    for s, x_ref in enumerate(x_refs):
        xt = x_ref[...].T
        ht = jnp.dot(w1t_ref[...], xt, preferred_element_type=jnp.float32)
        ht = ht + b1c_ref[...]
        ht = ht * jax.nn.sigmoid(ht)
        et = jnp.dot(w2r_ref[...], ht, preferred_element_type=jnp.float32)
        o_ref[s, 0, 0, :] = (et + b2_ref[0, 0])[0]


def _mlp(x, w1t, b1c, w2r, b2r, grid, block_off):
    return pl.pallas_call(
        _mlp_body,
        grid=(grid,),
        in_specs=[
            pl.BlockSpec(
                (ROWS, D),
                functools.partial(
                    lambda s, i: (block_off + i + s * grid, 0), s))
            for s in range(NSTREAM)
        ] + [
            pl.BlockSpec((D, D), lambda i: (0, 0)),
            pl.BlockSpec((D, 1), lambda i: (0, 0)),
            pl.BlockSpec((1, D), lambda i: (0, 0)),
            pl.BlockSpec((1, 1), lambda i: (0, 0)),
        ],
        out_specs=pl.BlockSpec((NSTREAM, 1, 1, ROWS), lambda i: (0, i, 0, 0)),
        out_shape=jax.ShapeDtypeStruct((NSTREAM, grid, 1, ROWS), jnp.float32),
        compiler_params=pltpu.CompilerParams(
            dimension_semantics=("arbitrary",),
        ),
    )(*([x] * NSTREAM), w1t, b1c, w2r, b2r)


NC = 2
NS = 16
NW = NC * NS
STRIP = 128

@functools.cache
def _make_segsum(n, gb_off):
    chunk = n // NW
    mesh = plsc.VectorSubcoreMesh(
        core_axis_name="c", subcore_axis_name="s",
        num_cores=NC, num_subcores=NS,
    )
    return functools.partial(
        pl.kernel,
        mesh=mesh,
        out_type=jax.ShapeDtypeStruct((NC, NUM_MOL), jnp.float32),
        scratch_types=[
            pltpu.VMEM((chunk,), jnp.float32),
            pltpu.VMEM((chunk + 16,), jnp.int32),
            pltpu.VMEM((NUM_MOL,), jnp.float32),
            pltpu.VMEM_SHARED((NS, NUM_MOL), jnp.float32),
            pltpu.VMEM((NS, STRIP), jnp.float32),
            pltpu.VMEM((STRIP,), jnp.float32),
            pltpu.SemaphoreType.DMA,
            pltpu.SemaphoreType.DMA,
        ],
        compiler_params=pltpu.CompilerParams(
            needs_layout_passes=False,
            disable_bounds_checks=True,
            disable_semaphore_checks=True,
        ),
    )(functools.partial(_segsum_body, chunk, gb_off))


def _segsum_body(CHUNK, gb_off, e_hbm, gb_hbm, out_hbm, e_buf, ids_buf, acc,
                 shared, red_buf, strip_buf, sem_e, sem_g):
    NVEC = CHUNK // 16
    cid = lax.axis_index("c")
    sid = lax.axis_index("s")
    wid = cid * NS + sid
    start = wid * CHUNK

    iota = lax.iota(jnp.int32, 16)
    sent_idx = jnp.where(iota < 8, iota, CHUNK + iota)
    plsc.store_scatter(ids_buf, [sent_idx], jnp.full((16,), -1, jnp.int32))

    cp_e = pltpu.async_copy(e_hbm.at[pl.ds(start, CHUNK)], e_buf, sem_e)
    cp_g = pltpu.async_copy(gb_hbm.at[pl.ds(gb_off + start, CHUNK)],
                            ids_buf.at[pl.ds(8, CHUNK)], sem_g)

    zeros16 = jnp.zeros((16,), jnp.float32)

    def zero_body(k, _):
        acc[pl.ds(k * 16, 16)] = zeros16
        return 0

    lax.fori_loop(0, NUM_MOL // 16, zero_body, 0)
    cp_e.wait()
    cp_g.wait()

    UNROLL = 5

    def _main(kk, carry):
        base0 = kk * (16 * UNROLL)
        for u in range(UNROLL):
            base = base0 + u * 16
            e_v = e_buf[pl.ds(base, 16)]
            ids = ids_buf[pl.ds(base + 8, 16)]
            prev = ids_buf[pl.ds(base + 7, 16)]
            nxt = ids_buf[pl.ds(base + 9, 16)]
            c = plsc.cumsum(e_v) + carry
            plsc.addupdate_scatter(acc, [ids], c, mask=ids != nxt)
            plsc.addupdate_scatter(acc, [ids], e_v - c, mask=ids != prev)
            carry = carry + jnp.sum(e_v)
        return carry

    lax.fori_loop(0, NVEC // UNROLL, _main, jnp.float32(0.0))

    pltpu.sync_copy(acc, shared.at[sid])
    plsc.subcore_barrier()

    @pl.when(sid < NUM_MOL // STRIP)
    def _():
        pltpu.sync_copy(shared.at[:, pl.ds(sid * STRIP, STRIP)], red_buf)

        def red_body(j, _):
            s = red_buf[0, pl.ds(j * 16, 16)]
            for t in range(1, NS):
                s = s + red_buf[t, pl.ds(j * 16, 16)]
            strip_buf[pl.ds(j * 16, 16)] = s
            return 0

        lax.fori_loop(0, STRIP // 16, red_body, 0)
        pltpu.sync_copy(strip_buf, out_hbm.at[cid, pl.ds(sid * STRIP, STRIP)])


N_A = 192000
N_B = N - N_A


def kernel(atoms_h, graph_batch, W1, b1, W2, b2):
    w1t = W1.T
    b1c = b1.reshape(D, 1).astype(jnp.float32)
    w2r = W2.reshape(1, D).astype(jnp.float32)
    b2r = b2.reshape(1, 1).astype(jnp.float32)
    gb = graph_batch.astype(jnp.int32)

    grid_a = N_A // (NSTREAM * ROWS)
    grid_b = N_B // (NSTREAM * ROWS)
    e_a = _mlp(atoms_h, w1t, b1c, w2r, b2r, grid_a, 0).reshape(N_A)
    e_b = _mlp(atoms_h, w1t, b1c, w2r, b2r, grid_b,
               NSTREAM * grid_a).reshape(N_B)
    p_a = _make_segsum(N_A, 0)(e_a, gb)
    p_b = _make_segsum(N_B, N_A)(e_b, gb)
    return p_a[0] + p_a[1] + p_b[0] + p_b[1]
